# baseline (device time: 2128252 ns/iter reference)
import os

import jax
import jax.numpy as jnp
from jax import lax
from jax.experimental import pallas as pl
from jax.experimental.pallas import tpu as pltpu

N_CHUNKS = 16
_NO_COPY = os.environ.get("DEBUG_NO_COPY", "0") == "1"
_NO_FWD = os.environ.get("DEBUG_NO_FWD", "0") == "1"
_NO_DIRECT = os.environ.get("DEBUG_NO_DIRECT", "0") == "1"


def kernel(x):
    m_per, n = x.shape
    half = m_per // 2
    chunk = half // N_CHUNKS

    n_copy = 2 * N_CHUNKS

    def body(x_ref, out_ref, copy_sems, d_send_sems, d_recv_sems,
             f_send_sems, f_recv_sems):
        my_x = lax.axis_index("x")
        my_y = lax.axis_index("y")
        y_nbr = (my_x, 1 - my_y)
        x_nbr = (1 - my_x, my_y)

        barrier_sem = pltpu.get_barrier_semaphore()
        for nbr in (y_nbr, x_nbr):
            pl.semaphore_signal(
                barrier_sem, inc=1,
                device_id=nbr, device_id_type=pl.DeviceIdType.MESH,
            )
        pl.semaphore_wait(barrier_sem, 2)

        local_copies = []
        for c in range(N_CHUNKS):
            big = m_per // N_CHUNKS
            off = c * big
            local_copies.append(pltpu.make_async_remote_copy(
                src_ref=x_ref.at[pl.ds(off, big)],
                dst_ref=out_ref.at[pl.ds(my_y * m_per + off, big)],
                send_sem=copy_sems.at[2 * c],
                recv_sem=copy_sems.at[2 * c + 1],
                device_id=(my_x, my_y),
                device_id_type=pl.DeviceIdType.MESH,
            ))

        recv_lo = (1 - my_y) * m_per + my_x * half
        fwd_recv_lo = (1 - my_y) * m_per + (1 - my_x) * half

        direct_sends = []
        for c in range(N_CHUNKS):
            off = c * chunk
            rdma = pltpu.make_async_remote_copy(
                src_ref=x_ref.at[pl.ds(my_x * half + off, chunk)],
                dst_ref=out_ref.at[pl.ds(my_y * m_per + my_x * half + off, chunk)],
                send_sem=d_send_sems.at[c],
                recv_sem=d_recv_sems.at[c],
                device_id=y_nbr,
                device_id_type=pl.DeviceIdType.MESH,
            )
            if not _NO_DIRECT:
                rdma.start()
                direct_sends.append(rdma)

        if not _NO_COPY:
            for cp in local_copies:
                cp.start()

        forwards = []
        for c in range(N_CHUNKS):
            off = c * chunk
            recv = pltpu.make_async_remote_copy(
                src_ref=x_ref.at[pl.ds(my_x * half + off, chunk)],
                dst_ref=out_ref.at[pl.ds(recv_lo + off, chunk)],
                send_sem=d_send_sems.at[c],
                recv_sem=d_recv_sems.at[c],
                device_id=y_nbr,
                device_id_type=pl.DeviceIdType.MESH,
            )
            if not _NO_DIRECT:
                recv.wait_recv()
            fwd = pltpu.make_async_remote_copy(
                src_ref=out_ref.at[pl.ds(recv_lo + off, chunk)],
                dst_ref=out_ref.at[pl.ds(recv_lo + off, chunk)],
                send_sem=f_send_sems.at[c],
                recv_sem=f_recv_sems.at[c],
                device_id=x_nbr,
                device_id_type=pl.DeviceIdType.MESH,
            )
            if not _NO_FWD:
                fwd.start()
                forwards.append(fwd)

        for c in range(N_CHUNKS):
            off = c * chunk
            recv = pltpu.make_async_remote_copy(
                src_ref=out_ref.at[pl.ds(recv_lo + off, chunk)],
                dst_ref=out_ref.at[pl.ds(fwd_recv_lo + off, chunk)],
                send_sem=f_send_sems.at[c],
                recv_sem=f_recv_sems.at[c],
                device_id=x_nbr,
                device_id_type=pl.DeviceIdType.MESH,
            )
            if not _NO_FWD:
                recv.wait_recv()
        for rdma in direct_sends:
            rdma.wait_send()
        for rdma in forwards:
            rdma.wait_send()
        if not _NO_COPY:
            for cp in local_copies:
                cp.wait()

    return pl.pallas_call(
        body,
        out_shape=jax.ShapeDtypeStruct((2 * m_per, n), x.dtype),
        in_specs=[pl.BlockSpec(memory_space=pl.ANY)],
        out_specs=pl.BlockSpec(memory_space=pl.ANY),
        scratch_shapes=[
            pltpu.SemaphoreType.DMA((2 * N_CHUNKS,)),
            pltpu.SemaphoreType.DMA((N_CHUNKS,)),
            pltpu.SemaphoreType.DMA((N_CHUNKS,)),
            pltpu.SemaphoreType.DMA((N_CHUNKS,)),
            pltpu.SemaphoreType.DMA((N_CHUNKS,)),
        ],
        compiler_params=pltpu.CompilerParams(collective_id=0),
    )(x)


# device time: 476631 ns/iter; 4.4652x vs baseline; 4.4652x over previous
import os

import jax
import jax.numpy as jnp
from jax import lax
from jax.experimental import pallas as pl
from jax.experimental.pallas import tpu as pltpu

N_CHUNKS = 16
_NO_COPY = os.environ.get("DEBUG_NO_COPY", "0") == "1"
_NO_FWD = os.environ.get("DEBUG_NO_FWD", "0") == "1"
_NO_DIRECT = os.environ.get("DEBUG_NO_DIRECT", "0") == "1"


def kernel(x):
    m_per, n = x.shape
    half = m_per // 2
    chunk = half // N_CHUNKS

    n_stage = 2 * N_CHUNKS
    rows = m_per // n_stage

    def body(x_ref, out_ref, stage_ref, in_sems, out_sems,
             d_send_sems, d_recv_sems, f_send_sems, f_recv_sems):
        my_x = lax.axis_index("x")
        my_y = lax.axis_index("y")
        y_nbr = (my_x, 1 - my_y)
        x_nbr = (1 - my_x, my_y)

        barrier_sem = pltpu.get_barrier_semaphore()
        for nbr in (y_nbr, x_nbr):
            pl.semaphore_signal(
                barrier_sem, inc=1,
                device_id=nbr, device_id_type=pl.DeviceIdType.MESH,
            )
        pl.semaphore_wait(barrier_sem, 2)

        recv_lo = (1 - my_y) * m_per + my_x * half
        fwd_recv_lo = (1 - my_y) * m_per + (1 - my_x) * half

        direct_sends = []
        for c in range(N_CHUNKS):
            off = c * chunk
            rdma = pltpu.make_async_remote_copy(
                src_ref=x_ref.at[pl.ds(my_x * half + off, chunk)],
                dst_ref=out_ref.at[pl.ds(my_y * m_per + my_x * half + off, chunk)],
                send_sem=d_send_sems.at[c],
                recv_sem=d_recv_sems.at[c],
                device_id=y_nbr,
                device_id_type=pl.DeviceIdType.MESH,
            )
            if not _NO_DIRECT:
                rdma.start()
                direct_sends.append(rdma)

        ins = [
            pltpu.make_async_copy(
                x_ref.at[pl.ds(s * rows, rows)],
                stage_ref.at[s % 2],
                in_sems.at[s % 2],
            )
            for s in range(n_stage)
        ]
        outs = [
            pltpu.make_async_copy(
                stage_ref.at[s % 2],
                out_ref.at[pl.ds(my_y * m_per + s * rows, rows)],
                out_sems.at[s % 2],
            )
            for s in range(n_stage)
        ]

        def drive_copy_stage(s):
            if _NO_COPY:
                return
            if s + 1 < n_stage:
                if s >= 1:
                    outs[s - 1].wait()
                ins[s + 1].start()
            ins[s].wait()
            outs[s].start()

        if not _NO_COPY:
            ins[0].start()

        forwards = []
        for c in range(N_CHUNKS):
            off = c * chunk
            drive_copy_stage(2 * c)
            drive_copy_stage(2 * c + 1)
            recv = pltpu.make_async_remote_copy(
                src_ref=x_ref.at[pl.ds(my_x * half + off, chunk)],
                dst_ref=out_ref.at[pl.ds(recv_lo + off, chunk)],
                send_sem=d_send_sems.at[c],
                recv_sem=d_recv_sems.at[c],
                device_id=y_nbr,
                device_id_type=pl.DeviceIdType.MESH,
            )
            if not _NO_DIRECT:
                recv.wait_recv()
            fwd = pltpu.make_async_remote_copy(
                src_ref=out_ref.at[pl.ds(recv_lo + off, chunk)],
                dst_ref=out_ref.at[pl.ds(recv_lo + off, chunk)],
                send_sem=f_send_sems.at[c],
                recv_sem=f_recv_sems.at[c],
                device_id=x_nbr,
                device_id_type=pl.DeviceIdType.MESH,
            )
            if not _NO_FWD:
                fwd.start()
                forwards.append(fwd)

        for c in range(N_CHUNKS):
            off = c * chunk
            recv = pltpu.make_async_remote_copy(
                src_ref=out_ref.at[pl.ds(recv_lo + off, chunk)],
                dst_ref=out_ref.at[pl.ds(fwd_recv_lo + off, chunk)],
                send_sem=f_send_sems.at[c],
                recv_sem=f_recv_sems.at[c],
                device_id=x_nbr,
                device_id_type=pl.DeviceIdType.MESH,
            )
            if not _NO_FWD:
                recv.wait_recv()
        if not _NO_COPY:
            outs[n_stage - 2].wait()
            outs[n_stage - 1].wait()
        for rdma in direct_sends:
            rdma.wait_send()
        for rdma in forwards:
            rdma.wait_send()

    return pl.pallas_call(
        body,
        out_shape=jax.ShapeDtypeStruct((2 * m_per, n), x.dtype),
        in_specs=[pl.BlockSpec(memory_space=pl.ANY)],
        out_specs=pl.BlockSpec(memory_space=pl.ANY),
        scratch_shapes=[
            pltpu.VMEM((2, rows, n), x.dtype),
            pltpu.SemaphoreType.DMA((2,)),
            pltpu.SemaphoreType.DMA((2,)),
            pltpu.SemaphoreType.DMA((N_CHUNKS,)),
            pltpu.SemaphoreType.DMA((N_CHUNKS,)),
            pltpu.SemaphoreType.DMA((N_CHUNKS,)),
            pltpu.SemaphoreType.DMA((N_CHUNKS,)),
        ],
        compiler_params=pltpu.CompilerParams(collective_id=0),
    )(x)


# device time: 452008 ns/iter; 4.7084x vs baseline; 1.0545x over previous
import os

import jax
import jax.numpy as jnp
from jax import lax
from jax.experimental import pallas as pl
from jax.experimental.pallas import tpu as pltpu

N_CHUNKS = 32
_NO_COPY = os.environ.get("DEBUG_NO_COPY", "0") == "1"
_NO_FWD = os.environ.get("DEBUG_NO_FWD", "0") == "1"
_NO_DIRECT = os.environ.get("DEBUG_NO_DIRECT", "0") == "1"


def kernel(x):
    m_per, n = x.shape
    half = m_per // 2
    chunk = half // N_CHUNKS

    n_stage = 2 * N_CHUNKS
    rows = m_per // n_stage

    def body(x_ref, out_ref, stage_ref, in_sems, out_sems,
             d_send_sems, d_recv_sems, f_send_sems, f_recv_sems):
        my_x = lax.axis_index("x")
        my_y = lax.axis_index("y")
        y_nbr = (my_x, 1 - my_y)
        x_nbr = (1 - my_x, my_y)

        barrier_sem = pltpu.get_barrier_semaphore()
        for nbr in (y_nbr, x_nbr):
            pl.semaphore_signal(
                barrier_sem, inc=1,
                device_id=nbr, device_id_type=pl.DeviceIdType.MESH,
            )
        pl.semaphore_wait(barrier_sem, 2)

        recv_lo = (1 - my_y) * m_per + my_x * half
        fwd_recv_lo = (1 - my_y) * m_per + (1 - my_x) * half

        direct_sends = []
        for c in range(N_CHUNKS):
            off = c * chunk
            rdma = pltpu.make_async_remote_copy(
                src_ref=x_ref.at[pl.ds(my_x * half + off, chunk)],
                dst_ref=out_ref.at[pl.ds(my_y * m_per + my_x * half + off, chunk)],
                send_sem=d_send_sems.at[c],
                recv_sem=d_recv_sems.at[c],
                device_id=y_nbr,
                device_id_type=pl.DeviceIdType.MESH,
            )
            if not _NO_DIRECT:
                rdma.start()
                direct_sends.append(rdma)

        ins = [
            pltpu.make_async_copy(
                x_ref.at[pl.ds(s * rows, rows)],
                stage_ref.at[s % 2],
                in_sems.at[s % 2],
            )
            for s in range(n_stage)
        ]
        outs = [
            pltpu.make_async_copy(
                stage_ref.at[s % 2],
                out_ref.at[pl.ds(my_y * m_per + s * rows, rows)],
                out_sems.at[s % 2],
            )
            for s in range(n_stage)
        ]

        def drive_copy_stage(s):
            if _NO_COPY:
                return
            if s + 1 < n_stage:
                if s >= 1:
                    outs[s - 1].wait()
                ins[s + 1].start()
            ins[s].wait()
            outs[s].start()

        if not _NO_COPY:
            ins[0].start()

        forwards = []
        for c in range(N_CHUNKS):
            off = c * chunk
            drive_copy_stage(2 * c)
            drive_copy_stage(2 * c + 1)
            recv = pltpu.make_async_remote_copy(
                src_ref=x_ref.at[pl.ds(my_x * half + off, chunk)],
                dst_ref=out_ref.at[pl.ds(recv_lo + off, chunk)],
                send_sem=d_send_sems.at[c],
                recv_sem=d_recv_sems.at[c],
                device_id=y_nbr,
                device_id_type=pl.DeviceIdType.MESH,
            )
            if not _NO_DIRECT:
                recv.wait_recv()
            fwd = pltpu.make_async_remote_copy(
                src_ref=out_ref.at[pl.ds(recv_lo + off, chunk)],
                dst_ref=out_ref.at[pl.ds(recv_lo + off, chunk)],
                send_sem=f_send_sems.at[c],
                recv_sem=f_recv_sems.at[c],
                device_id=x_nbr,
                device_id_type=pl.DeviceIdType.MESH,
            )
            if not _NO_FWD:
                fwd.start()
                forwards.append(fwd)

        for c in range(N_CHUNKS):
            off = c * chunk
            recv = pltpu.make_async_remote_copy(
                src_ref=out_ref.at[pl.ds(recv_lo + off, chunk)],
                dst_ref=out_ref.at[pl.ds(fwd_recv_lo + off, chunk)],
                send_sem=f_send_sems.at[c],
                recv_sem=f_recv_sems.at[c],
                device_id=x_nbr,
                device_id_type=pl.DeviceIdType.MESH,
            )
            if not _NO_FWD:
                recv.wait_recv()
        if not _NO_COPY:
            outs[n_stage - 2].wait()
            outs[n_stage - 1].wait()
        for rdma in direct_sends:
            rdma.wait_send()
        for rdma in forwards:
            rdma.wait_send()

    return pl.pallas_call(
        body,
        out_shape=jax.ShapeDtypeStruct((2 * m_per, n), x.dtype),
        in_specs=[pl.BlockSpec(memory_space=pl.ANY)],
        out_specs=pl.BlockSpec(memory_space=pl.ANY),
        scratch_shapes=[
            pltpu.VMEM((2, rows, n), x.dtype),
            pltpu.SemaphoreType.DMA((2,)),
            pltpu.SemaphoreType.DMA((2,)),
            pltpu.SemaphoreType.DMA((N_CHUNKS,)),
            pltpu.SemaphoreType.DMA((N_CHUNKS,)),
            pltpu.SemaphoreType.DMA((N_CHUNKS,)),
            pltpu.SemaphoreType.DMA((N_CHUNKS,)),
        ],
        compiler_params=pltpu.CompilerParams(collective_id=0),
    )(x)
